# Initial kernel scaffold; baseline (speedup 1.0000x reference)
import jax
import jax.numpy as jnp
from jax.experimental import pallas as pl


def kernel(x, edge_index, W1, W2):
    raise NotImplementedError("write your pallas kernel here")


# speedup vs baseline: 1.0230x; 1.0230x over previous
import functools

import jax
import jax.numpy as jnp
from jax.experimental import pallas as pl
from jax.experimental.pallas import tpu as pltpu

N = 10000
IN_FEATS = 128
N_HID = 64
OUT_FEATS = 40
ALPHA = 1.0

ROW_BLK = 1000


def _layer1_body(x_ref, agg_ref, w_ref, o_ref):
    h = (x_ref[...] + ALPHA * agg_ref[...]) @ w_ref[...]
    nrm = jnp.sqrt(jnp.sum(h * h, axis=1, keepdims=True))
    h = h / jnp.clip(nrm, 1e-12)
    o_ref[...] = jnp.maximum(h, 0.0)


def _layer2_body(h_ref, agg_ref, w_ref, o_ref):
    o_ref[...] = (h_ref[...] + ALPHA * agg_ref[...]) @ w_ref[...]


def _dense_layer1(x, agg, W1):
    grid = (N // ROW_BLK,)
    return pl.pallas_call(
        _layer1_body,
        grid=grid,
        in_specs=[
            pl.BlockSpec((ROW_BLK, IN_FEATS), lambda i: (i, 0)),
            pl.BlockSpec((ROW_BLK, IN_FEATS), lambda i: (i, 0)),
            pl.BlockSpec((IN_FEATS, N_HID), lambda i: (0, 0)),
        ],
        out_specs=pl.BlockSpec((ROW_BLK, N_HID), lambda i: (i, 0)),
        out_shape=jax.ShapeDtypeStruct((N, N_HID), jnp.float32),
    )(x, agg, W1)


def _dense_layer2(h, agg, W2):
    grid = (N // ROW_BLK,)
    return pl.pallas_call(
        _layer2_body,
        grid=grid,
        in_specs=[
            pl.BlockSpec((ROW_BLK, N_HID), lambda i: (i, 0)),
            pl.BlockSpec((ROW_BLK, N_HID), lambda i: (i, 0)),
            pl.BlockSpec((N_HID, OUT_FEATS), lambda i: (0, 0)),
        ],
        out_specs=pl.BlockSpec((ROW_BLK, OUT_FEATS), lambda i: (i, 0)),
        out_shape=jax.ShapeDtypeStruct((N, OUT_FEATS), jnp.float32),
    )(h, agg, W2)


def _seg_max(feat, src, dst):
    msgs = feat[src]
    agg = jax.ops.segment_max(msgs, dst, num_segments=feat.shape[0])
    return jnp.where(jnp.isfinite(agg), agg, 0.0)


def kernel(x, edge_index, W1, W2):
    src = edge_index[0].astype(jnp.int32)
    dst = edge_index[1].astype(jnp.int32)
    agg1 = _seg_max(x, src, dst)
    h = _dense_layer1(x, agg1, W1)
    agg2 = _seg_max(h, src, dst)
    return _dense_layer2(h, agg2, W2)


# speedup vs baseline: 2.5010x; 2.4448x over previous
import functools

import jax
import jax.numpy as jnp
from jax import lax
from jax.experimental import pallas as pl
from jax.experimental.pallas import tpu as pltpu
from jax.experimental.pallas import tpu_sc as plsc

N = 10000
E = 320000
IN_FEATS = 128
N_HID = 64
OUT_FEATS = 40
ALPHA = 1.0

NC, NS, L = 2, 16, 16
NW = NC * NS
ROWS = 320
NPAD = NW * ROWS
CAP = 12288
ECHUNK = 8000
NECHUNKS = E // ECHUNK
GCH = 128

ROW_BLK = 1000

_NEG = float("-inf")


def _wid():
    return lax.axis_index("s") * NC + lax.axis_index("c")


def _mesh():
    return plsc.VectorSubcoreMesh(
        core_axis_name="c", subcore_axis_name="s",
        num_cores=NC, num_subcores=NS)


def _fill(ref, start, num, value):
    splat = jnp.full((L,), value, ref.dtype)

    def body(i, _):
        ref[pl.ds(start + i * L, L)] = splat
        return 0

    lax.fori_loop(0, num // L, body, 0)


def _agg_phase(feat_hbm, src_list, dstl_list, acc, rows, sem, ne, d):
    nvec = d // L
    ngr = (ne + GCH - 1) // GCH

    def chunk_body(g, _):
        idx = src_list.at[pl.ds(g * GCH, GCH)]
        pltpu.async_copy(feat_hbm.at[idx], rows, sem).wait()

        def grp_body(q, _):
            dvec = dstl_list[pl.ds(g * GCH + q * L, L)]
            for j in range(L):
                base = dvec[j] * d
                r = q * L + j
                for f in range(nvec):
                    a = acc[pl.ds(base + f * L, L)]
                    v = rows[r, pl.ds(f * L, L)]
                    acc[pl.ds(base + f * L, L)] = jnp.maximum(a, v)
            return 0

        lax.fori_loop(0, GCH // L, grp_body, 0)
        return 0

    lax.fori_loop(0, ngr, chunk_body, 0)


def _finalize(acc, d):
    zeros = jnp.zeros((L,), jnp.float32)

    def body(i, _):
        a = acc[pl.ds(i * L, L)]
        acc[pl.ds(i * L, L)] = jnp.where(a > jnp.float32(-3e38), a, zeros)
        return 0

    lax.fori_loop(0, ROWS * d // L, body, 0)


def _sc_layer1_body(x_hbm, srce_hbm, dste_hbm, agg_hbm, slists_hbm,
                    dlists_hbm, cnts_hbm, src_list, dstl_list, acc,
                    ebuf_s, ebuf_d, rows, cnt_buf, sem):
    wid = _wid()
    lo = wid * ROWS

    _fill(src_list, 0, CAP, 0)
    _fill(dstl_list, 0, CAP, ROWS)
    _fill(acc, 0, (ROWS + 1) * IN_FEATS, _NEG)

    def chunk_body(c, off):
        pltpu.sync_copy(srce_hbm.at[pl.ds(c * ECHUNK, ECHUNK)], ebuf_s)
        pltpu.sync_copy(dste_hbm.at[pl.ds(c * ECHUNK, ECHUNK)], ebuf_d)

        def vec_body(i, off):
            sv = ebuf_s[pl.ds(i * L, L)]
            dv = ebuf_d[pl.ds(i * L, L)]
            m = (dv >= lo) & (dv < lo + ROWS)
            pos = off + plsc.cumsum(m.astype(jnp.int32)) - 1
            plsc.store_scatter(src_list, [pos], sv, mask=m)
            plsc.store_scatter(dstl_list, [pos], dv - lo, mask=m)
            return pos[L - 1] + 1

        return lax.fori_loop(0, ECHUNK // L, vec_body, off)

    ne = lax.fori_loop(0, NECHUNKS, chunk_body, jnp.int32(0))

    pltpu.sync_copy(src_list, slists_hbm.at[wid])
    pltpu.sync_copy(dstl_list, dlists_hbm.at[wid])
    cnt_buf[...] = jnp.full((L,), ne, jnp.int32)
    pltpu.sync_copy(cnt_buf, cnts_hbm.at[wid])

    _agg_phase(x_hbm, src_list, dstl_list, acc, rows, sem, ne, IN_FEATS)
    _finalize(acc, IN_FEATS)
    pltpu.sync_copy(acc.at[pl.ds(0, ROWS * IN_FEATS)],
                    agg_hbm.at[pl.ds(lo * IN_FEATS, ROWS * IN_FEATS)])


def _sc_layer2_body(h_hbm, slists_hbm, dlists_hbm, cnts_hbm, agg_hbm,
                    src_list, dstl_list, acc, rows, cnt_buf, sem):
    wid = _wid()
    lo = wid * ROWS

    _fill(acc, 0, (ROWS + 1) * N_HID, _NEG)
    pltpu.sync_copy(slists_hbm.at[wid], src_list)
    pltpu.sync_copy(dlists_hbm.at[wid], dstl_list)
    pltpu.sync_copy(cnts_hbm.at[wid], cnt_buf)
    ne = jnp.max(cnt_buf[...])

    _agg_phase(h_hbm, src_list, dstl_list, acc, rows, sem, ne, N_HID)
    _finalize(acc, N_HID)
    pltpu.sync_copy(acc.at[pl.ds(0, ROWS * N_HID)],
                    agg_hbm.at[pl.ds(lo * N_HID, ROWS * N_HID)])


@jax.jit
def _sc_layer1(x, srce, dste):
    return pl.kernel(
        _sc_layer1_body,
        out_type=(
            jax.ShapeDtypeStruct((NPAD * IN_FEATS,), jnp.float32),
            jax.ShapeDtypeStruct((NW, CAP), jnp.int32),
            jax.ShapeDtypeStruct((NW, CAP), jnp.int32),
            jax.ShapeDtypeStruct((NW, L), jnp.int32),
        ),
        mesh=_mesh(),
        compiler_params=pltpu.CompilerParams(needs_layout_passes=False),
        scratch_types=[
            pltpu.VMEM((CAP,), jnp.int32),
            pltpu.VMEM((CAP,), jnp.int32),
            pltpu.VMEM(((ROWS + 1) * IN_FEATS,), jnp.float32),
            pltpu.VMEM((ECHUNK,), jnp.int32),
            pltpu.VMEM((ECHUNK,), jnp.int32),
            pltpu.VMEM((GCH, IN_FEATS), jnp.float32),
            pltpu.VMEM((L,), jnp.int32),
            pltpu.SemaphoreType.DMA,
        ],
    )(x, srce, dste)


@jax.jit
def _sc_layer2(h, slists, dlists, cnts):
    return pl.kernel(
        _sc_layer2_body,
        out_type=jax.ShapeDtypeStruct((NPAD * N_HID,), jnp.float32),
        mesh=_mesh(),
        compiler_params=pltpu.CompilerParams(
            needs_layout_passes=False, use_tc_tiling_on_sc=False),
        scratch_types=[
            pltpu.VMEM((CAP,), jnp.int32),
            pltpu.VMEM((CAP,), jnp.int32),
            pltpu.VMEM(((ROWS + 1) * N_HID,), jnp.float32),
            pltpu.VMEM((GCH, N_HID), jnp.float32),
            pltpu.VMEM((L,), jnp.int32),
            pltpu.SemaphoreType.DMA,
        ],
    )(h, slists, dlists, cnts)


def _layer1_tc_body(x_ref, agg_ref, w_ref, o_ref):
    h = (x_ref[...] + ALPHA * agg_ref[...]) @ w_ref[...]
    nrm = jnp.sqrt(jnp.sum(h * h, axis=1, keepdims=True))
    h = h / jnp.clip(nrm, 1e-12)
    o_ref[...] = jnp.maximum(h, 0.0)


def _layer2_tc_body(h_ref, agg_ref, w_ref, o_ref):
    o_ref[...] = (h_ref[...] + ALPHA * agg_ref[...]) @ w_ref[...]


def _dense_layer1(x, agg, W1):
    return pl.pallas_call(
        _layer1_tc_body,
        grid=(N // ROW_BLK,),
        in_specs=[
            pl.BlockSpec((ROW_BLK, IN_FEATS), lambda i: (i, 0)),
            pl.BlockSpec((ROW_BLK, IN_FEATS), lambda i: (i, 0)),
            pl.BlockSpec((IN_FEATS, N_HID), lambda i: (0, 0)),
        ],
        out_specs=pl.BlockSpec((ROW_BLK, N_HID), lambda i: (i, 0)),
        out_shape=jax.ShapeDtypeStruct((N, N_HID), jnp.float32),
    )(x, agg, W1)


def _dense_layer2(h, agg, W2):
    return pl.pallas_call(
        _layer2_tc_body,
        grid=(N // ROW_BLK,),
        in_specs=[
            pl.BlockSpec((ROW_BLK, N_HID), lambda i: (i, 0)),
            pl.BlockSpec((ROW_BLK, N_HID), lambda i: (i, 0)),
            pl.BlockSpec((N_HID, OUT_FEATS), lambda i: (0, 0)),
        ],
        out_specs=pl.BlockSpec((ROW_BLK, OUT_FEATS), lambda i: (i, 0)),
        out_shape=jax.ShapeDtypeStruct((N, OUT_FEATS), jnp.float32),
    )(h, agg, W2)


def kernel(x, edge_index, W1, W2):
    srce = edge_index[0].astype(jnp.int32)
    dste = edge_index[1].astype(jnp.int32)
    agg1_flat, slists, dlists, cnts = _sc_layer1(x, srce, dste)
    agg1 = agg1_flat.reshape(NPAD, IN_FEATS)[:N]
    h = _dense_layer1(x, agg1, W1)
    agg2_flat = _sc_layer2(h, slists, dlists, cnts)
    agg2 = agg2_flat.reshape(NPAD, N_HID)[:N]
    return _dense_layer2(h, agg2, W2)
